# baseline (device time: 16803 ns/iter reference)
import jax
import jax.numpy as jnp
from jax import lax
from jax.experimental import pallas as pl
from jax.experimental.pallas import tpu as pltpu

CHUNKS = (64, 128, 128, 96, 48, 32, 16)


def kernel(x):
    m, n = x.shape
    half = n // 2
    hrows = m // 2
    C = len(CHUNKS)
    offs = []
    o = 0
    for s in CHUNKS:
        offs.append(o)
        o += s
    assert o == hrows
    out_dtype = jnp.bfloat16

    def body(x_ref, out_ref, send_buf, ysend_sems, yrecv_sems,
             xsend_sems, xrecv_sems):
        my_x = lax.axis_index("x")
        my_y = lax.axis_index("y")
        peer_y = (my_x, 1 - my_y)
        peer_x = (1 - my_x, my_y)
        my_half = my_x * hrows

        barrier_sem = pltpu.get_barrier_semaphore()
        for nbr in (peer_y, peer_x):
            pl.semaphore_signal(
                barrier_sem, inc=1,
                device_id=nbr, device_id_type=pl.DeviceIdType.MESH,
            )

        def stage(c):
            sl = pl.ds(offs[c], CHUNKS[c])

            @pl.when(my_y == 0)
            def _():
                send_buf[sl, :] = (
                    x_ref[pl.ds(my_half + offs[c], CHUNKS[c]), half:]
                    .astype(out_dtype))

            @pl.when(my_y == 1)
            def _():
                send_buf[sl, :] = (
                    x_ref[pl.ds(my_half + offs[c], CHUNKS[c]), :half]
                    .astype(out_dtype))

        stage(0)
        pl.semaphore_wait(barrier_sem, 2)

        rdmas_y = []
        for c in range(C):
            if c > 0:
                stage(c)
            rdma = pltpu.make_async_remote_copy(
                src_ref=send_buf.at[pl.ds(offs[c], CHUNKS[c])],
                dst_ref=out_ref.at[pl.ds(my_y * m + my_half + offs[c],
                                         CHUNKS[c])],
                send_sem=ysend_sems.at[c],
                recv_sem=yrecv_sems.at[c],
                device_id=peer_y,
                device_id_type=pl.DeviceIdType.MESH,
            )
            rdma.start()
            rdmas_y.append(rdma)

        def store_local(c):
            sl2 = pl.ds(2 * offs[c], 2 * CHUNKS[c])

            @pl.when(my_y == 0)
            def _():
                out_ref[sl2, :] = x_ref[sl2, :half].astype(out_dtype)

            @pl.when(my_y == 1)
            def _():
                out_ref[pl.ds(m + 2 * offs[c], 2 * CHUNKS[c]), :] = (
                    x_ref[sl2, half:].astype(out_dtype))

        rdmas_x = []
        for c in range(C):
            rdmas_y[c].wait_recv()
            row0 = (1 - my_y) * m + my_half + offs[c]
            fwd = pltpu.make_async_remote_copy(
                src_ref=out_ref.at[pl.ds(row0, CHUNKS[c])],
                dst_ref=out_ref.at[pl.ds(row0, CHUNKS[c])],
                send_sem=xsend_sems.at[c],
                recv_sem=xrecv_sems.at[c],
                device_id=peer_x,
                device_id_type=pl.DeviceIdType.MESH,
            )
            fwd.start()
            rdmas_x.append(fwd)
            store_local(c)

        for c in range(C):
            rdmas_x[c].wait_recv()
        for c in range(C):
            rdmas_y[c].wait_send()
            rdmas_x[c].wait_send()

    return pl.pallas_call(
        body,
        out_shape=jax.ShapeDtypeStruct((2 * m, half), out_dtype),
        in_specs=[pl.BlockSpec(memory_space=pltpu.VMEM)],
        out_specs=pl.BlockSpec(memory_space=pltpu.VMEM),
        scratch_shapes=[
            pltpu.VMEM((hrows, half), out_dtype),
            pltpu.SemaphoreType.DMA((C,)),
            pltpu.SemaphoreType.DMA((C,)),
            pltpu.SemaphoreType.DMA((C,)),
            pltpu.SemaphoreType.DMA((C,)),
        ],
        compiler_params=pltpu.CompilerParams(collective_id=0),
    )(x)


# device time: 16127 ns/iter; 1.0419x vs baseline; 1.0419x over previous
import jax
import jax.numpy as jnp
from jax import lax
from jax.experimental import pallas as pl
from jax.experimental.pallas import tpu as pltpu

C = 16


def kernel(x):
    m, n = x.shape
    half = n // 2
    hrows = m // 2
    chs = hrows // C
    out_dtype = jnp.bfloat16

    def body(x_ref, out_ref, send_buf, ysend_sems, yrecv_sems,
             xsend_sems, xrecv_sems):
        my_x = lax.axis_index("x")
        my_y = lax.axis_index("y")
        peer_y = (my_x, 1 - my_y)
        peer_x = (1 - my_x, my_y)

        barrier_sem = pltpu.get_barrier_semaphore()
        for nbr in (peer_y, peer_x):
            pl.semaphore_signal(
                barrier_sem, inc=1,
                device_id=nbr, device_id_type=pl.DeviceIdType.MESH,
            )
        pl.semaphore_wait(barrier_sem, 2)

        my_half_off = my_x * hrows

        rdmas_y = []
        for c in range(C):
            row0 = my_half_off + c * chs

            @pl.when(my_y == 0)
            def _(row0=row0, c=c):
                send_buf[pl.ds(c * chs, chs), :] = (
                    x_ref[pl.ds(row0, chs), half:].astype(out_dtype))

            @pl.when(my_y == 1)
            def _(row0=row0, c=c):
                send_buf[pl.ds(c * chs, chs), :] = (
                    x_ref[pl.ds(row0, chs), :half].astype(out_dtype))

            rdma = pltpu.make_async_remote_copy(
                src_ref=send_buf.at[pl.ds(c * chs, chs)],
                dst_ref=out_ref.at[pl.ds(my_y * m + row0, chs)],
                send_sem=ysend_sems.at[c],
                recv_sem=yrecv_sems.at[c],
                device_id=peer_y,
                device_id_type=pl.DeviceIdType.MESH,
            )
            rdma.start()
            rdmas_y.append(rdma)

        @pl.when(my_y == 0)
        def _():
            out_ref[pl.ds(0, m), :] = x_ref[:, :half].astype(out_dtype)

        @pl.when(my_y == 1)
        def _():
            out_ref[pl.ds(m, m), :] = x_ref[:, half:].astype(out_dtype)

        rdmas_x = []
        for c in range(C):
            rdmas_y[c].wait_recv()
            row0 = (1 - my_y) * m + my_half_off + c * chs
            fwd = pltpu.make_async_remote_copy(
                src_ref=out_ref.at[pl.ds(row0, chs)],
                dst_ref=out_ref.at[pl.ds(row0, chs)],
                send_sem=xsend_sems.at[c],
                recv_sem=xrecv_sems.at[c],
                device_id=peer_x,
                device_id_type=pl.DeviceIdType.MESH,
            )
            fwd.start()
            rdmas_x.append(fwd)

        for c in range(C):
            rdmas_x[c].wait_recv()
        for c in range(C):
            rdmas_y[c].wait_send()
            rdmas_x[c].wait_send()

    return pl.pallas_call(
        body,
        out_shape=jax.ShapeDtypeStruct((2 * m, half), out_dtype),
        in_specs=[pl.BlockSpec(memory_space=pltpu.VMEM)],
        out_specs=pl.BlockSpec(memory_space=pltpu.VMEM),
        scratch_shapes=[
            pltpu.VMEM((hrows, half), out_dtype),
            pltpu.SemaphoreType.DMA((C,)),
            pltpu.SemaphoreType.DMA((C,)),
            pltpu.SemaphoreType.DMA((C,)),
            pltpu.SemaphoreType.DMA((C,)),
        ],
        compiler_params=pltpu.CompilerParams(collective_id=0),
    )(x)
